# baseline (speedup 1.0000x reference)
import functools

import jax
import jax.numpy as jnp
from jax import lax
from jax.experimental import pallas as pl
from jax.experimental.pallas import tpu as pltpu
from jax.experimental.pallas import tpu_sc as plsc

_N_ENT = 10000
_N_REL = 200
_H = 8
_LANES = 16
_NC, _NS = 2, 16
_NW = _NC * _NS
_NPW = (_N_ENT + _NW - 1) // _NW
_CR = _N_REL * _LANES
_SSTAGE = 344


def _mm_kernel(a_ref, b_ref, o_ref):
    o_ref[...] = jnp.dot(a_ref[...], b_ref[...],
                         preferred_element_type=jnp.float32)


def _mm(a, b, bm=512, bn=512):
    m, k = a.shape
    k2, n = b.shape
    assert k == k2
    bm = min(bm, m)
    bn = min(bn, n)
    grid = (pl.cdiv(m, bm), pl.cdiv(n, bn))
    return pl.pallas_call(
        _mm_kernel,
        grid=grid,
        in_specs=[
            pl.BlockSpec((bm, k), lambda i, j: (i, 0)),
            pl.BlockSpec((k, bn), lambda i, j: (0, j)),
        ],
        out_specs=pl.BlockSpec((bm, bn), lambda i, j: (i, j)),
        out_shape=jax.ShapeDtypeStruct((m, n), jnp.float32),
    )(a, b)


def _sc_weighted_segsum(ypad, alpha16, t_s, r_s, starts, upad):
    fdim = _H * upad
    nchunk = upad // _LANES
    mesh = plsc.VectorSubcoreMesh(core_axis_name="c", subcore_axis_name="s")
    zeros_row = jnp.zeros((_CR,), jnp.float32)

    @functools.partial(
        pl.kernel, mesh=mesh,
        compiler_params=pltpu.CompilerParams(needs_layout_passes=False),
        out_type=[jax.ShapeDtypeStruct((_N_ENT * fdim,), jnp.float32),
                  jax.ShapeDtypeStruct((_N_ENT * _CR,), jnp.float32)],
        scratch_types=[
            pltpu.VMEM((fdim,), jnp.float32),
            pltpu.VMEM((_CR,), jnp.float32),
            pltpu.VMEM((16, fdim), jnp.float32),
            pltpu.VMEM((24,), jnp.int32),
            pltpu.VMEM((24,), jnp.int32),
            pltpu.VMEM((256,), jnp.float32),
            pltpu.VMEM((_SSTAGE,), jnp.int32),
            pltpu.SemaphoreType.DMA,
        ],
    )
    def sc_kernel(y_h, al_h, t_h, r_h, st_h, z_h, out_t_h, out_c_h,
                  acc_v, crow_v, rows_v, tind_v, rind_v, alpha_v,
                  starts_v, sem):
        wid = lax.axis_index("s") * _NC + lax.axis_index("c")
        n_lo = wid * _NPW
        n_hi = jnp.minimum(n_lo + _NPW, _N_ENT)
        sbase = (n_lo // 8) * 8
        pltpu.sync_copy(st_h.at[pl.ds(sbase, _SSTAGE)], starts_v)
        iota = lax.iota(jnp.int32, 16)

        def _scal(ref, i):
            vec = ref[pl.ds(i, 16)]
            return jnp.max(jnp.where(iota == 0, vec, 0))

        def ent_body(n, carry):
            i_loc = n - sbase
            st = _scal(starts_v, i_loc)
            en = _scal(starts_v, i_loc + 1)
            pltpu.sync_copy(z_h.at[pl.ds(0, fdim)], acc_v)
            pltpu.sync_copy(z_h, crow_v)
            nb = (en - st + 15) // 16

            def batch_body(k, carry2):
                eb = st + k * 16
                b8 = (eb // 8) * 8
                o = eb - b8
                pltpu.sync_copy(t_h.at[pl.ds(b8, 24)], tind_v)
                pltpu.sync_copy(r_h.at[pl.ds(b8, 24)], rind_v)
                pltpu.sync_copy(al_h.at[pl.ds(eb * 16, 256)], alpha_v)
                tvec = tind_v[pl.ds(o, 16)]
                pltpu.async_copy(y_h.at[tvec], rows_v, sem).wait()
                rvec = rind_v[pl.ds(o, 16)]
                nv = jnp.minimum(en - eb, 16)

                def edge_body(v, carry3):
                    avec = alpha_v[pl.ds(v * 16, 16)]
                    rsp = rvec.at[jnp.full((16,), v, jnp.int32)].get(
                        mode="promise_in_bounds")
                    rsc = jnp.max(rsp)
                    coff = rsc * 16
                    crow_v[pl.ds(coff, 16)] = crow_v[pl.ds(coff, 16)] + avec
                    for hh in range(_H):
                        w = avec.at[jnp.full((16,), hh, jnp.int32)].get(
                            mode="promise_in_bounds")
                        for c in range(nchunk):
                            off = hh * upad + c * _LANES
                            y = rows_v[v, pl.ds(off, _LANES)]
                            acc_v[pl.ds(off, _LANES)] = (
                                acc_v[pl.ds(off, _LANES)] + w * y)
                    return carry3

                lax.fori_loop(0, nv, edge_body, carry2)
                return carry2

            lax.fori_loop(0, nb, batch_body, 0)
            pltpu.sync_copy(acc_v, out_t_h.at[pl.ds(n * fdim, fdim)])
            pltpu.sync_copy(crow_v, out_c_h.at[pl.ds(n * _CR, _CR)])
            return carry

        lax.fori_loop(n_lo, n_hi, ent_body, 0)

    return sc_kernel(ypad, alpha16, t_s, r_s, starts, zeros_row)


def _layer(h_s, r_s, t_s, tpad_s, rpad_s, starts, ent, rel, W, a, Wr, upad):
    n_ent, fdim = ent.shape
    n_rel, gdim = rel.shape
    num_heads, units = a.shape

    Xh = _mm(ent, W[:fdim])
    Xt = _mm(ent, W[fdim:2 * fdim])
    Xr = _mm(rel, W[2 * fdim:])

    a_flat = a.reshape(-1)
    ph = (Xh * a_flat).reshape(n_ent, num_heads, units).sum(-1)
    pt = (Xt * a_flat).reshape(n_ent, num_heads, units).sum(-1)
    pr = (Xr * a_flat).reshape(n_rel, num_heads, units).sum(-1)

    b = jax.nn.leaky_relu(ph[h_s] + pt[t_s] + pr[r_s],
                          negative_slope=0.2)
    m = jax.ops.segment_max(b, h_s, num_segments=n_ent,
                            indices_are_sorted=True)
    m = jnp.where(jnp.isfinite(m), m, 0.0)
    ex = jnp.exp(b - m[h_s])
    s = jax.ops.segment_sum(ex, h_s, num_segments=n_ent,
                            indices_are_sorted=True)
    alpha = ex / (s[h_s] + 1e-9)
    S0 = jax.ops.segment_sum(alpha, h_s, num_segments=n_ent,
                             indices_are_sorted=True)

    e_tot = h_s.shape[0]
    alpha16 = jnp.zeros((e_tot + 16, 16), jnp.float32)
    alpha16 = alpha16.at[:e_tot, :num_heads].set(alpha).reshape(-1)
    Xt3 = Xt.reshape(n_ent, num_heads, units)
    ypad = jnp.pad(Xt3, ((0, 0), (0, 0), (0, upad - units)))
    ypad = ypad.reshape(n_ent, num_heads * upad)
    out_t_pad, C = _sc_weighted_segsum(ypad, alpha16, tpad_s, rpad_s,
                                       starts, upad)
    out_t = out_t_pad.reshape(n_ent, num_heads, upad)[:, :, :units]

    C3 = C.reshape(n_ent, n_rel, 16)
    Xr3 = Xr.reshape(n_rel, num_heads, units)
    out_r = []
    for i in range(num_heads):
        out_r.append(_mm(C3[:, :, i], Xr3[:, i, :]))
    out_r = jnp.stack(out_r, axis=1)

    out = S0[:, :, None] * Xh.reshape(n_ent, num_heads, units) + out_t + out_r
    rel_out = _mm(rel, Wr, bm=256, bn=256)
    return out, rel_out


def kernel(h_index, r_index, t_index, entity_embeddings, relation_embeddings,
           W0, a0, Wr0, W1, a1, Wr1, Wd, bd):
    order = jnp.arange(h_index.shape[0])
    h_s = h_index[order]
    t_s = t_index[order]
    r_s = r_index[order]
    e_tot = h_s.shape[0]
    tpad_s = jnp.zeros((e_tot + 32,), jnp.int32).at[:e_tot].set(t_s)
    rpad_s = jnp.zeros((e_tot + 32,), jnp.int32).at[:e_tot].set(r_s)
    starts = jnp.searchsorted(h_s, jnp.arange(_N_ENT + 1),
                              side="left").astype(jnp.int32)
    starts = jnp.full((10040,), e_tot, jnp.int32).at[:_N_ENT + 1].set(starts)

    out0, rel0 = _layer(h_s, r_s, t_s, tpad_s, rpad_s, starts,
                        entity_embeddings, relation_embeddings,
                        W0, a0, Wr0, upad=160)
    ent_feat = jax.nn.elu(out0).reshape(_N_ENT, -1)

    out1, rel1 = _layer(h_s, r_s, t_s, tpad_s, rpad_s, starts,
                        ent_feat, rel0, W1, a1, Wr1, upad=304)
    feats1 = jax.nn.elu(out1)
    ent_out = feats1.sum(axis=1)
    ent_out = ent_out + _mm(entity_embeddings, Wd, bm=512, bn=512) + bd
    return ent_out, rel1


# speedup vs baseline: 12.1584x; 12.1584x over previous
import functools

import jax
import jax.numpy as jnp
from jax import lax
from jax.experimental import pallas as pl
from jax.experimental.pallas import tpu as pltpu
from jax.experimental.pallas import tpu_sc as plsc

_N_ENT = 10000
_N_REL = 200
_H = 8
_LANES = 16
_NC, _NS = 2, 16
_NW = _NC * _NS
_NPW = (_N_ENT + _NW - 1) // _NW
_CR = _N_REL * _LANES
_SSTAGE = 344


def _mm_kernel(a_ref, b_ref, o_ref):
    o_ref[...] = jnp.dot(a_ref[...], b_ref[...],
                         preferred_element_type=jnp.float32)


def _mm(a, b, bm=512, bn=512):
    m, k = a.shape
    k2, n = b.shape
    assert k == k2
    bm = min(bm, m)
    bn = min(bn, n)
    grid = (pl.cdiv(m, bm), pl.cdiv(n, bn))
    return pl.pallas_call(
        _mm_kernel,
        grid=grid,
        in_specs=[
            pl.BlockSpec((bm, k), lambda i, j: (i, 0)),
            pl.BlockSpec((k, bn), lambda i, j: (0, j)),
        ],
        out_specs=pl.BlockSpec((bm, bn), lambda i, j: (i, j)),
        out_shape=jax.ShapeDtypeStruct((m, n), jnp.float32),
    )(a, b)


def _sc_weighted_segsum(ypad, alpha16, t_s, r_s, starts, upad):
    fdim = _H * upad
    nchunk = upad // _LANES
    mesh = plsc.VectorSubcoreMesh(core_axis_name="c", subcore_axis_name="s")
    zeros_row = jnp.zeros((_CR,), jnp.float32)

    @functools.partial(
        pl.kernel, mesh=mesh,
        compiler_params=pltpu.CompilerParams(needs_layout_passes=False),
        out_type=[jax.ShapeDtypeStruct((_N_ENT * fdim,), jnp.float32),
                  jax.ShapeDtypeStruct((_N_ENT * _CR,), jnp.float32)],
        scratch_types=[
            pltpu.VMEM((fdim,), jnp.float32),
            pltpu.VMEM((_CR,), jnp.float32),
            pltpu.VMEM((16, fdim), jnp.float32),
            pltpu.VMEM((24,), jnp.int32),
            pltpu.VMEM((24,), jnp.int32),
            pltpu.VMEM((256,), jnp.float32),
            pltpu.VMEM((2048,), jnp.float32),
            pltpu.VMEM((_SSTAGE,), jnp.int32),
            pltpu.SemaphoreType.DMA,
        ],
    )
    def sc_kernel(y_h, al_h, t_h, r_h, st_h, z_h, out_t_h, out_c_h,
                  acc_v, crow_v, rows_v, tind_v, rind_v, alpha_v,
                  wsp_v, starts_v, sem):
        wid = lax.axis_index("s") * _NC + lax.axis_index("c")
        n_lo = wid * _NPW
        n_hi = jnp.minimum(n_lo + _NPW, _N_ENT)
        sbase = (n_lo // 8) * 8
        pltpu.sync_copy(st_h.at[pl.ds(sbase, _SSTAGE)], starts_v)
        iota = lax.iota(jnp.int32, 16)

        def _scal(ref, i):
            vec = ref[pl.ds(i, 16)]
            return jnp.max(jnp.where(iota == 0, vec, 0))

        def ent_body(n, carry):
            i_loc = n - sbase
            st = _scal(starts_v, i_loc)
            en = _scal(starts_v, i_loc + 1)
            pltpu.sync_copy(z_h.at[pl.ds(0, fdim)], acc_v)
            pltpu.sync_copy(z_h, crow_v)
            nb = (en - st + 15) // 16

            def batch_body(k, carry2):
                eb = st + k * 16
                b8 = (eb // 8) * 8
                o = eb - b8
                pltpu.sync_copy(t_h.at[pl.ds(b8, 24)], tind_v)
                pltpu.sync_copy(r_h.at[pl.ds(b8, 24)], rind_v)
                pltpu.sync_copy(al_h.at[pl.ds(eb * 16, 256)], alpha_v)
                tvec = tind_v[pl.ds(o, 16)]
                pltpu.async_copy(y_h.at[tvec], rows_v, sem).wait()
                rvec = rind_v[pl.ds(o, 16)]
                nv = jnp.minimum(en - eb, 16)

                for v in range(16):
                    avec = alpha_v[pl.ds(v * 16, 16)]
                    avm = jnp.where(v < nv, avec, 0.0)
                    rsp = rvec.at[jnp.full((16,), v, jnp.int32)].get(
                        mode="promise_in_bounds")
                    coff = jnp.max(rsp) * 16
                    crow_v[pl.ds(coff, 16)] = crow_v[pl.ds(coff, 16)] + avm
                    for hh in range(_H):
                        w = avm.at[jnp.full((16,), hh, jnp.int32)].get(
                            mode="promise_in_bounds")
                        wsp_v[pl.ds(v * 128 + hh * 16, 16)] = w

                def chunk_body(j, carry3):
                    hh16 = (j // nchunk) * 16
                    base = j * 16
                    accv = acc_v[pl.ds(base, 16)]
                    for v in range(16):
                        w = wsp_v[pl.ds(v * 128 + hh16, 16)]
                        y = rows_v[v, pl.ds(base, 16)]
                        accv = accv + w * y
                    acc_v[pl.ds(base, 16)] = accv
                    return carry3

                lax.fori_loop(0, _H * nchunk, chunk_body, carry2)
                return carry2

            lax.fori_loop(0, nb, batch_body, 0)
            pltpu.sync_copy(acc_v, out_t_h.at[pl.ds(n * fdim, fdim)])
            pltpu.sync_copy(crow_v, out_c_h.at[pl.ds(n * _CR, _CR)])
            return carry

        lax.fori_loop(n_lo, n_hi, ent_body, 0)

    return sc_kernel(ypad, alpha16, t_s, r_s, starts, zeros_row)


def _layer(h_s, r_s, t_s, tpad_s, rpad_s, starts, ent, rel, W, a, Wr, upad):
    n_ent, fdim = ent.shape
    n_rel, gdim = rel.shape
    num_heads, units = a.shape

    Xh = _mm(ent, W[:fdim])
    Xt = _mm(ent, W[fdim:2 * fdim])
    Xr = _mm(rel, W[2 * fdim:])

    a_flat = a.reshape(-1)
    ph = (Xh * a_flat).reshape(n_ent, num_heads, units).sum(-1)
    pt = (Xt * a_flat).reshape(n_ent, num_heads, units).sum(-1)
    pr = (Xr * a_flat).reshape(n_rel, num_heads, units).sum(-1)

    b = jax.nn.leaky_relu(ph[h_s] + pt[t_s] + pr[r_s],
                          negative_slope=0.2)
    m = jax.ops.segment_max(b, h_s, num_segments=n_ent,
                            indices_are_sorted=True)
    m = jnp.where(jnp.isfinite(m), m, 0.0)
    ex = jnp.exp(b - m[h_s])
    s = jax.ops.segment_sum(ex, h_s, num_segments=n_ent,
                            indices_are_sorted=True)
    alpha = ex / (s[h_s] + 1e-9)
    S0 = jax.ops.segment_sum(alpha, h_s, num_segments=n_ent,
                             indices_are_sorted=True)

    e_tot = h_s.shape[0]
    alpha16 = jnp.zeros((e_tot + 16, 16), jnp.float32)
    alpha16 = alpha16.at[:e_tot, :num_heads].set(alpha).reshape(-1)
    Xt3 = Xt.reshape(n_ent, num_heads, units)
    ypad = jnp.pad(Xt3, ((0, 0), (0, 0), (0, upad - units)))
    ypad = ypad.reshape(n_ent, num_heads * upad)
    out_t_pad, C = _sc_weighted_segsum(ypad, alpha16, tpad_s, rpad_s,
                                       starts, upad)
    out_t = out_t_pad.reshape(n_ent, num_heads, upad)[:, :, :units]

    C4 = jnp.transpose(C.reshape(n_ent, n_rel, 16), (2, 0, 1))
    Xr3 = Xr.reshape(n_rel, num_heads, units)
    out_r = []
    for i in range(num_heads):
        out_r.append(_mm(C4[i], Xr3[:, i, :]))
    out_r = jnp.stack(out_r, axis=1)

    out = S0[:, :, None] * Xh.reshape(n_ent, num_heads, units) + out_t + out_r
    rel_out = _mm(rel, Wr, bm=256, bn=256)
    return out, rel_out


def kernel(h_index, r_index, t_index, entity_embeddings, relation_embeddings,
           W0, a0, Wr0, W1, a1, Wr1, Wd, bd):
    order = jnp.argsort(h_index)
    h_s = h_index[order]
    t_s = t_index[order]
    r_s = r_index[order]
    e_tot = h_s.shape[0]
    tpad_s = jnp.zeros((e_tot + 32,), jnp.int32).at[:e_tot].set(t_s)
    rpad_s = jnp.zeros((e_tot + 32,), jnp.int32).at[:e_tot].set(r_s)
    starts = jnp.searchsorted(h_s, jnp.arange(_N_ENT + 1),
                              side="left").astype(jnp.int32)
    starts = jnp.full((10040,), e_tot, jnp.int32).at[:_N_ENT + 1].set(starts)

    out0, rel0 = _layer(h_s, r_s, t_s, tpad_s, rpad_s, starts,
                        entity_embeddings, relation_embeddings,
                        W0, a0, Wr0, upad=160)
    ent_feat = jax.nn.elu(out0).reshape(_N_ENT, -1)

    out1, rel1 = _layer(h_s, r_s, t_s, tpad_s, rpad_s, starts,
                        ent_feat, rel0, W1, a1, Wr1, upad=304)
    feats1 = jax.nn.elu(out1)
    ent_out = feats1.sum(axis=1)
    ent_out = ent_out + _mm(entity_embeddings, Wd, bm=512, bn=512) + bd
    return ent_out, rel1
